# baseline (speedup 1.0000x reference)
import functools

import jax
import jax.numpy as jnp
from jax import lax
from jax.experimental import pallas as pl
from jax.experimental.pallas import tpu as pltpu
from jax.experimental.pallas import tpu_sc as plsc

ROWS, D = 8192, 768
NC, NS = 2, 16
NW = NC * NS
ROWS_PER = ROWS // NW

_mesh = plsc.VectorSubcoreMesh(core_axis_name="c", subcore_axis_name="s")


@functools.partial(
    pl.kernel,
    mesh=_mesh,
    out_type=jax.ShapeDtypeStruct((ROWS, D), jnp.float32),
)
def _sc_copy(w_hbm, out_hbm):
    wid = lax.axis_index("s") * NC + lax.axis_index("c")
    base = wid * ROWS_PER
    pltpu.sync_copy(w_hbm.at[pl.ds(base, ROWS_PER)], out_hbm.at[pl.ds(base, ROWS_PER)])


def kernel(x, W):
    del x
    return _sc_copy(W)


# speedup vs baseline: 20.5255x; 20.5255x over previous
import functools

import jax
import jax.numpy as jnp
from jax import lax
from jax.experimental import pallas as pl
from jax.experimental.pallas import tpu as pltpu
from jax.experimental.pallas import tpu_sc as plsc

ROWS, D = 8192, 768
NC, NS = 2, 16
NW = NC * NS
ROWS_PER = ROWS // NW
CHUNK = 64
NCHUNK = ROWS_PER // CHUNK

_mesh = plsc.VectorSubcoreMesh(core_axis_name="c", subcore_axis_name="s")


@functools.partial(
    pl.kernel,
    mesh=_mesh,
    out_type=jax.ShapeDtypeStruct((ROWS, D), jnp.float32),
    scratch_types=[
        pltpu.VMEM((CHUNK, D), jnp.float32),
        pltpu.VMEM((CHUNK, D), jnp.float32),
        pltpu.SemaphoreType.DMA,
        pltpu.SemaphoreType.DMA,
        pltpu.SemaphoreType.DMA,
        pltpu.SemaphoreType.DMA,
    ],
)
def _sc_copy(w_hbm, out_hbm, buf0, buf1, g0, g1, s0, s1):
    wid = lax.axis_index("s") * NC + lax.axis_index("c")
    base = wid * ROWS_PER
    bufs = (buf0, buf1)
    gsem = (g0, g1)
    ssem = (s0, s1)

    gathers = [None] * NCHUNK
    scatters = [None] * NCHUNK
    gathers[0] = pltpu.async_copy(
        w_hbm.at[pl.ds(base, CHUNK)], bufs[0], gsem[0]
    )
    for i in range(NCHUNK):
        b = i % 2
        gathers[i].wait()
        scatters[i] = pltpu.async_copy(
            bufs[b], out_hbm.at[pl.ds(base + i * CHUNK, CHUNK)], ssem[b]
        )
        if i + 1 < NCHUNK:
            nb = (i + 1) % 2
            if i >= 1:
                scatters[i - 1].wait()
            gathers[i + 1] = pltpu.async_copy(
                w_hbm.at[pl.ds(base + (i + 1) * CHUNK, CHUNK)], bufs[nb], gsem[nb]
            )
    scatters[NCHUNK - 2].wait()
    scatters[NCHUNK - 1].wait()


def kernel(x, W):
    del x
    return _sc_copy(W)
